# Initial kernel scaffold; baseline (speedup 1.0000x reference)
import jax
import jax.numpy as jnp
from jax.experimental import pallas as pl


def kernel(x, edge_index, W1, a_src1, a_dst1, b1, W2, a_src2, a_dst2, b2):
    raise NotImplementedError("write your pallas kernel here")


# speedup vs baseline: 21.1414x; 21.1414x over previous
import functools

import jax
import jax.numpy as jnp
from jax import lax
from jax.experimental import pallas as pl
from jax.experimental.pallas import tpu as pltpu
from jax.experimental.pallas import tpu_sc as plsc

N = 10000
D = 128
E = 320000

RW = 144
NP = 10016
RPT = NP // 16
KC = 128
NW = 32
CPW = 81
ET = NW * CPW * KC
BN = 2504
NEG = -1.0e30

_f32 = jnp.float32
_i32 = jnp.int32


def _head(h, asrc, adst, hext_ref, as_ref, ad_ref):
    ones = jnp.ones((NP, 1), _f32)
    zer = jnp.zeros((NP, RW - D - 1), _f32)
    hext_ref[...] = jnp.concatenate([h, ones, zer], axis=1)
    row = lax.broadcasted_iota(_i32, (1, NP), 1)
    valid = row < N
    a_s = jnp.sum(h * asrc, axis=1).reshape(1, NP)
    a_d = jnp.sum(h * adst, axis=1).reshape(1, NP)
    as_ref[...] = jnp.where(valid, a_s, NEG)
    ad_ref[...] = jnp.where(valid, a_d, NEG)


def _tc_first_body(x_ref, w_ref, asrc_ref, adst_ref, hext_ref, as_ref, ad_ref):
    h = jnp.dot(x_ref[...], w_ref[...], preferred_element_type=_f32)
    _head(h, asrc_ref[...], adst_ref[...], hext_ref, as_ref, ad_ref)


def _combine(part_ref):
    p = part_ref[...]
    ssum = p[0] + p[1]
    numer = ssum[:, :D]
    denom = jnp.sum(ssum[:, D:RW], axis=1, keepdims=True)
    return numer / (denom + 1e-16)


def _tc_mid_body(part_ref, b_ref, w_ref, asrc_ref, adst_ref,
                 hext_ref, as_ref, ad_ref):
    h0 = _combine(part_ref) + b_ref[...]
    h0 = jnp.where(h0 >= 0.0, h0, 0.01 * h0)
    h = jnp.dot(h0, w_ref[...], preferred_element_type=_f32)
    _head(h, asrc_ref[...], adst_ref[...], hext_ref, as_ref, ad_ref)


def _tc_last_body(part_ref, b_ref, out_ref):
    out_ref[...] = _combine(part_ref) + b_ref[...]


_HEAD_OUT = [
    jax.ShapeDtypeStruct((NP, RW), _f32),
    jax.ShapeDtypeStruct((1, NP), _f32),
    jax.ShapeDtypeStruct((1, NP), _f32),
]

_tc_first = pl.pallas_call(_tc_first_body, out_shape=_HEAD_OUT)
_tc_mid = pl.pallas_call(_tc_mid_body, out_shape=_HEAD_OUT)
_tc_last = pl.pallas_call(
    _tc_last_body, out_shape=[jax.ShapeDtypeStruct((NP, D), _f32)])


_mesh = plsc.VectorSubcoreMesh(core_axis_name="c", subcore_axis_name="s")


@functools.partial(
    pl.kernel,
    out_type=jax.ShapeDtypeStruct((2 * NP, RW), _f32),
    mesh=_mesh,
    compiler_params=pltpu.CompilerParams(needs_layout_passes=False,
                                         use_tc_tiling_on_sc=False),
    scratch_types=[
        pltpu.VMEM((KC,), _i32),
        pltpu.VMEM((KC,), _i32),
        pltpu.VMEM((KC, RW), _f32),
        pltpu.VMEM((KC,), _f32),
        pltpu.VMEM((NP,), _f32),
        pltpu.VMEM((NP,), _f32),
        pltpu.VMEM_SHARED((NP, RW), _f32),
        pltpu.SemaphoreType.DMA,
    ],
)
def _edge_kernel(hext, asarr, adarr, src, dst, zrows, out,
                 srcv, dstv, rowsv, wv, asv, adv, accum, sem):
    c = lax.axis_index("c")
    s = lax.axis_index("s")
    wid = s * 2 + c

    pltpu.sync_copy(zrows.at[pl.ds(s * RPT, RPT)],
                    accum.at[pl.ds(s * RPT, RPT)])
    pltpu.sync_copy(asarr, asv)
    pltpu.sync_copy(adarr, adv)
    plsc.subcore_barrier()

    def chunk_body(i, carry):
        base = (wid * CPW + i) * KC
        pltpu.sync_copy(src.at[pl.ds(base, KC)], srcv)
        pltpu.sync_copy(dst.at[pl.ds(base, KC)], dstv)
        pltpu.async_copy(hext.at[srcv], rowsv, sem).wait()
        for j in range(KC // 16):
            sl = pl.ds(j * 16, 16)
            e = (plsc.load_gather(asv, [srcv[sl]])
                 + plsc.load_gather(adv, [dstv[sl]]))
            e = jnp.where(e >= 0.0, e, 0.2 * e)
            wv[sl] = jnp.exp(e)

        def scale_body(j, carry2):
            wsp = plsc.load_gather(wv, [jnp.full((16,), j, _i32)])
            for r in range(RW // 16):
                sl2 = pl.ds(r * 16, 16)
                rowsv[j, sl2] = rowsv[j, sl2] * wsp
            return carry2

        lax.fori_loop(0, KC, scale_body, 0)
        pltpu.sync_copy(rowsv, accum.at[dstv], add=True)
        return carry

    lax.fori_loop(0, CPW, chunk_body, 0)

    plsc.subcore_barrier()
    pltpu.sync_copy(accum.at[pl.ds(s * RPT, RPT)],
                    out.at[pl.ds(c * NP + s * RPT, RPT)])


def kernel(x, edge_index, W1, a_src1, a_dst1, b1, W2, a_src2, a_dst2, b2):
    loop = jnp.arange(N, dtype=_i32)
    padi = jnp.full((ET - E - N,), N, dtype=_i32)
    src = jnp.concatenate([edge_index[0].astype(_i32), loop, padi])
    dst = jnp.concatenate([edge_index[1].astype(_i32), loop, padi])

    xp = jnp.zeros((NP, D), _f32).at[:N].set(x)
    zrows = jnp.zeros((NP, RW), _f32)

    asrc1 = a_src1.reshape(1, D)
    adst1 = a_dst1.reshape(1, D)
    asrc2 = a_src2.reshape(1, D)
    adst2 = a_dst2.reshape(1, D)

    hext1, as1, ad1 = _tc_first(xp, W1, asrc1, adst1)
    part1 = _edge_kernel(hext1, as1.reshape(NP), ad1.reshape(NP),
                         src, dst, zrows)
    hext2, as2, ad2 = _tc_mid(part1.reshape(2, NP, RW), b1.reshape(1, D),
                              W2, asrc2, adst2)
    part2 = _edge_kernel(hext2, as2.reshape(NP), ad2.reshape(NP),
                         src, dst, zrows)
    (outp,) = _tc_last(part2.reshape(2, NP, RW), b2.reshape(1, D))
    return outp[:N]


# speedup vs baseline: 34.4136x; 1.6278x over previous
import functools

import jax
import jax.numpy as jnp
from jax import lax
from jax.experimental import pallas as pl
from jax.experimental.pallas import tpu as pltpu
from jax.experimental.pallas import tpu_sc as plsc

N = 10000
D = 128
E = 320000

RW = 144
NP = 10016
RPT = NP // 16
KC = 128
NW = 32
CPW = 81
ET = NW * CPW * KC
BN = 2504
NEG = -1.0e30

_f32 = jnp.float32
_i32 = jnp.int32


def _head(h, asrc, adst, hext_ref, as_ref, ad_ref):
    ones = jnp.ones((NP, 1), _f32)
    zer = jnp.zeros((NP, RW - D - 1), _f32)
    hext_ref[...] = jnp.concatenate([h, ones, zer], axis=1)
    row = lax.broadcasted_iota(_i32, (1, NP), 1)
    valid = row < N
    a_s = jnp.sum(h * asrc, axis=1).reshape(1, NP)
    a_d = jnp.sum(h * adst, axis=1).reshape(1, NP)
    as_ref[...] = jnp.where(valid, a_s, NEG)
    ad_ref[...] = jnp.where(valid, a_d, NEG)


def _tc_first_body(x_ref, w_ref, asrc_ref, adst_ref, hext_ref, as_ref, ad_ref):
    h = jnp.dot(x_ref[...], w_ref[...], preferred_element_type=_f32)
    _head(h, asrc_ref[...], adst_ref[...], hext_ref, as_ref, ad_ref)


def _combine(part_ref):
    p = part_ref[...]
    ssum = p[0] + p[1]
    numer = ssum[:, :D]
    denom = jnp.sum(ssum[:, D:RW], axis=1, keepdims=True)
    return numer / (denom + 1e-16)


def _tc_mid_body(part_ref, b_ref, w_ref, asrc_ref, adst_ref,
                 hext_ref, as_ref, ad_ref):
    h0 = _combine(part_ref) + b_ref[...]
    h0 = jnp.where(h0 >= 0.0, h0, 0.01 * h0)
    h = jnp.dot(h0, w_ref[...], preferred_element_type=_f32)
    _head(h, asrc_ref[...], adst_ref[...], hext_ref, as_ref, ad_ref)


def _tc_last_body(part_ref, b_ref, out_ref):
    out_ref[...] = _combine(part_ref) + b_ref[...]


_HEAD_OUT = [
    jax.ShapeDtypeStruct((NP, RW), _f32),
    jax.ShapeDtypeStruct((1, NP), _f32),
    jax.ShapeDtypeStruct((1, NP), _f32),
]

_tc_first = pl.pallas_call(_tc_first_body, out_shape=_HEAD_OUT)
_tc_mid = pl.pallas_call(_tc_mid_body, out_shape=_HEAD_OUT)
_tc_last = pl.pallas_call(
    _tc_last_body, out_shape=[jax.ShapeDtypeStruct((NP, D), _f32)])


_mesh = plsc.VectorSubcoreMesh(core_axis_name="c", subcore_axis_name="s")


@functools.partial(
    pl.kernel,
    out_type=jax.ShapeDtypeStruct((2 * NP, RW), _f32),
    mesh=_mesh,
    compiler_params=pltpu.CompilerParams(needs_layout_passes=False,
                                         use_tc_tiling_on_sc=False),
    scratch_types=[
        pltpu.VMEM((KC,), _i32),
        pltpu.VMEM((KC,), _i32),
        pltpu.VMEM((KC,), _i32),
        pltpu.VMEM((KC,), _i32),
        pltpu.VMEM((KC, RW), _f32),
        pltpu.VMEM((KC, RW), _f32),
        pltpu.VMEM((KC,), _i32),
        pltpu.VMEM((KC,), _f32),
        pltpu.VMEM((KC,), _f32),
        pltpu.VMEM((KC,), _f32),
        pltpu.VMEM((KC,), _f32),
        pltpu.VMEM((KC,), _f32),
        pltpu.VMEM_SHARED((NP, RW), _f32),
        pltpu.SemaphoreType.DMA,
        pltpu.SemaphoreType.DMA,
        pltpu.SemaphoreType.DMA,
        pltpu.SemaphoreType.DMA,
    ],
)
def _edge_kernel(hext, asarr, adarr, src, dst, zrows, out,
                 srcv0, dstv0, srcv1, dstv1, rows0, rows1, sidx, wv,
                 asg0, adg0, asg1, adg1, accum, semg0, semg1, sems, semi):
    c = lax.axis_index("c")
    s = lax.axis_index("s")
    wid = s * 2 + c
    gbase = wid * CPW

    pltpu.sync_copy(zrows.at[pl.ds(s * RPT, RPT)],
                    accum.at[pl.ds(s * RPT, RPT)])
    plsc.subcore_barrier()

    bufs = ((srcv0, dstv0, rows0, semg0, asg0, adg0),
            (srcv1, dstv1, rows1, semg1, asg1, adg1))

    def idx_issue(g, p):
        base = (gbase + g) * KC
        pltpu.async_copy(src.at[pl.ds(base, KC)], bufs[p][0], semi)
        pltpu.async_copy(dst.at[pl.ds(base, KC)], bufs[p][1], semi)

    def idx_wait(g, p):
        base = (gbase + g) * KC
        pltpu.make_async_copy(src.at[pl.ds(base, KC)], bufs[p][0], semi).wait()
        pltpu.make_async_copy(dst.at[pl.ds(base, KC)], bufs[p][1], semi).wait()

    def gather_issue(p):
        sv, dv, rows, semg, asg, adg = bufs[p]
        pltpu.async_copy(hext.at[sv], rows, semg)
        pltpu.async_copy(asarr.at[sv], asg, semg)
        pltpu.async_copy(adarr.at[dv], adg, semg)

    def gather_wait(p):
        sv, dv, rows, semg, asg, adg = bufs[p]
        pltpu.make_async_copy(hext.at[sv], rows, semg).wait()
        pltpu.make_async_copy(asarr.at[sv], asg, semg).wait()
        pltpu.make_async_copy(adarr.at[dv], adg, semg).wait()

    def scatter_wait(p):
        pltpu.make_async_copy(bufs[p][2], accum.at[sidx], sems).wait()

    def process(g, p, first=False, next_idx=True, next_gather=True):
        sv, dv, rows, _, asg, adg = bufs[p]
        if not first:
            scatter_wait(1 - p)
        if next_gather:
            idx_wait(g + 1, 1 - p)
            gather_issue(1 - p)
        gather_wait(p)
        for j in range(KC // 16):
            sl = pl.ds(j * 16, 16)
            e = asg[sl] + adg[sl]
            e = jnp.where(e >= 0.0, e, 0.2 * e)
            wv[sl] = jnp.exp(e)
            sidx[sl] = dv[sl]
        if next_idx:
            idx_issue(g + 2, p)

        @plsc.parallel_loop(0, KC, unroll=2)
        def _scale(j):
            wsp = plsc.load_gather(wv, [jnp.full((16,), j, _i32)])
            for r in range(RW // 16):
                sl2 = pl.ds(r * 16, 16)
                rows[j, sl2] = rows[j, sl2] * wsp

        pltpu.async_copy(rows, accum.at[sidx], sems, add=True)

    idx_issue(0, 0)
    idx_wait(0, 0)
    gather_issue(0)
    idx_issue(1, 1)
    process(0, 0, first=True)

    def body(i2, carry):
        g = 1 + 2 * i2
        process(g, 1)
        process(g + 1, 0)
        return carry

    lax.fori_loop(0, (CPW - 3) // 2, body, 0)
    process(CPW - 2, 1, next_idx=False)
    process(CPW - 1, 0, next_idx=False, next_gather=False)
    scatter_wait(0)

    plsc.subcore_barrier()
    pltpu.sync_copy(accum.at[pl.ds(s * RPT, RPT)],
                    out.at[pl.ds(c * NP + s * RPT, RPT)])


def kernel(x, edge_index, W1, a_src1, a_dst1, b1, W2, a_src2, a_dst2, b2):
    loop = jnp.arange(N, dtype=_i32)
    padi = jnp.full((ET - E - N,), N, dtype=_i32)
    src = jnp.concatenate([edge_index[0].astype(_i32), loop, padi])
    dst = jnp.concatenate([edge_index[1].astype(_i32), loop, padi])

    xp = jnp.zeros((NP, D), _f32).at[:N].set(x)
    zrows = jnp.zeros((NP, RW), _f32)

    asrc1 = a_src1.reshape(1, D)
    adst1 = a_dst1.reshape(1, D)
    asrc2 = a_src2.reshape(1, D)
    adst2 = a_dst2.reshape(1, D)

    hext1, as1, ad1 = _tc_first(xp, W1, asrc1, adst1)
    part1 = _edge_kernel(hext1, as1.reshape(NP), ad1.reshape(NP),
                         src, dst, zrows)
    hext2, as2, ad2 = _tc_mid(part1.reshape(2, NP, RW), b1.reshape(1, D),
                              W2, asrc2, adst2)
    part2 = _edge_kernel(hext2, as2.reshape(NP), ad2.reshape(NP),
                         src, dst, zrows)
    (outp,) = _tc_last(part2.reshape(2, NP, RW), b2.reshape(1, D))
    return outp[:N]
